# Initial kernel scaffold; baseline (speedup 1.0000x reference)
import jax
import jax.numpy as jnp
from jax.experimental import pallas as pl


def kernel(x_td, Win_qcd, Wout_qdc, emb_qkc):
    raise NotImplementedError("write your pallas kernel here")


# speedup vs baseline: 1.3076x; 1.3076x over previous
import jax
import jax.numpy as jnp
from jax.experimental import pallas as pl

NUM_Q = 8
INPUT_DIM = 512
CODE_DIM = 256
KSIZE = 2048
T = 8192

BT = 256


def _rvq_kernel(x_ref, win_ref, wout_ref, emb_ref, out_ref, codes_ref):
    res = x_ref[...]
    out = jnp.zeros_like(res)
    idx_rows = []
    for q in range(NUM_Q):
        win_q = win_ref[q]
        emb_q = emb_ref[q]
        wout_q = wout_ref[q]

        xp = jax.lax.dot_general(
            res, win_q, (((1,), (1,)), ((), ())),
            preferred_element_type=jnp.float32)
        x_sq = jnp.sum(xp * xp, axis=-1, keepdims=True)
        e_sq = jnp.sum(emb_q * emb_q, axis=-1)[None, :]
        cross = jax.lax.dot_general(
            xp, emb_q, (((1,), (1,)), ((), ())),
            preferred_element_type=jnp.float32)
        dist = x_sq - 2 * cross + e_sq

        minv = jnp.min(dist, axis=-1, keepdims=True)
        iota = jax.lax.broadcasted_iota(jnp.int32, (BT, KSIZE), 1)
        idx = jnp.min(jnp.where(dist == minv, iota, KSIZE),
                      axis=-1, keepdims=True)
        idx_rows.append(idx)

        onehot = (iota == idx).astype(jnp.float32)
        quant = jax.lax.dot_general(
            onehot, emb_q, (((1,), (0,)), ((), ())),
            preferred_element_type=jnp.float32)
        dec = jax.lax.dot_general(
            quant, wout_q, (((1,), (1,)), ((), ())),
            preferred_element_type=jnp.float32)
        res = res - dec
        out = out + dec

    out_ref[...] = out
    codes_ref[...] = jnp.concatenate(
        [r.reshape(1, BT) for r in idx_rows], axis=0)


def kernel(x_td, Win_qcd, Wout_qdc, emb_qkc):
    grid = (T // BT,)
    out_td, codes_qt = pl.pallas_call(
        _rvq_kernel,
        grid=grid,
        in_specs=[
            pl.BlockSpec((BT, INPUT_DIM), lambda i: (i, 0)),
            pl.BlockSpec((NUM_Q, CODE_DIM, INPUT_DIM), lambda i: (0, 0, 0)),
            pl.BlockSpec((NUM_Q, INPUT_DIM, CODE_DIM), lambda i: (0, 0, 0)),
            pl.BlockSpec((NUM_Q, KSIZE, CODE_DIM), lambda i: (0, 0, 0)),
        ],
        out_specs=(
            pl.BlockSpec((BT, INPUT_DIM), lambda i: (i, 0)),
            pl.BlockSpec((NUM_Q, BT), lambda i: (0, i)),
        ),
        out_shape=(
            jax.ShapeDtypeStruct((T, INPUT_DIM), jnp.float32),
            jax.ShapeDtypeStruct((NUM_Q, T), jnp.int32),
        ),
    )(x_td, Win_qcd, Wout_qdc, emb_qkc)
    return out_td, codes_qt


# speedup vs baseline: 1.4115x; 1.0795x over previous
import jax
import jax.numpy as jnp
from jax.experimental import pallas as pl

NUM_Q = 8
INPUT_DIM = 512
CODE_DIM = 256
KSIZE = 2048
T = 8192

BT = 256


def _esq_kernel(emb2_ref, esq_ref):
    e2 = emb2_ref[...]
    esq_ref[...] = 0.25 * jnp.sum(e2 * e2, axis=-1)


def _rvq_kernel(x_ref, win_ref, wout_ref, emb2_ref, esq_ref,
                out_ref, codes_ref):
    res = x_ref[...]
    out = jnp.zeros_like(res)
    iota = jax.lax.broadcasted_iota(jnp.int32, (BT, KSIZE), 1)
    idx_rows = []
    for q in range(NUM_Q):
        win_q = win_ref[q]
        emb2_q = emb2_ref[q]
        wout_q = wout_ref[q]

        xp = jax.lax.dot_general(
            res, win_q, (((1,), (1,)), ((), ())),
            preferred_element_type=jnp.float32)
        x_sq = jnp.sum(xp * xp, axis=-1, keepdims=True)
        cross2 = jax.lax.dot_general(
            xp, emb2_q, (((1,), (1,)), ((), ())),
            preferred_element_type=jnp.float32)
        dist = (x_sq - cross2) + esq_ref[q][None, :]

        di = jax.lax.bitcast_convert_type(dist, jnp.int32)
        base = jax.lax.bitcast_convert_type(x_sq, jnp.int32)
        key = (di - base) * KSIZE + iota
        minkey = jnp.min(key, axis=-1, keepdims=True)
        idx = jnp.bitwise_and(minkey, KSIZE - 1)
        idx_rows.append(idx)

        onehot = (iota == idx).astype(jnp.float32)
        quant2 = jax.lax.dot_general(
            onehot, emb2_q, (((1,), (0,)), ((), ())),
            preferred_element_type=jnp.float32)
        dec = 0.5 * jax.lax.dot_general(
            quant2, wout_q, (((1,), (1,)), ((), ())),
            preferred_element_type=jnp.float32)
        res = res - dec
        out = out + dec

    out_ref[...] = out
    codes_ref[...] = jnp.concatenate(
        [r.reshape(1, BT) for r in idx_rows], axis=0)


def kernel(x_td, Win_qcd, Wout_qdc, emb_qkc):
    emb2 = emb_qkc * 2.0
    esq_qk = pl.pallas_call(
        _esq_kernel,
        out_shape=jax.ShapeDtypeStruct((NUM_Q, KSIZE), jnp.float32),
    )(emb2)

    grid = (T // BT,)
    out_td, codes_qt = pl.pallas_call(
        _rvq_kernel,
        grid=grid,
        in_specs=[
            pl.BlockSpec((BT, INPUT_DIM), lambda i: (i, 0)),
            pl.BlockSpec((NUM_Q, CODE_DIM, INPUT_DIM), lambda i: (0, 0, 0)),
            pl.BlockSpec((NUM_Q, INPUT_DIM, CODE_DIM), lambda i: (0, 0, 0)),
            pl.BlockSpec((NUM_Q, KSIZE, CODE_DIM), lambda i: (0, 0, 0)),
            pl.BlockSpec((NUM_Q, KSIZE), lambda i: (0, 0)),
        ],
        out_specs=(
            pl.BlockSpec((BT, INPUT_DIM), lambda i: (i, 0)),
            pl.BlockSpec((NUM_Q, BT), lambda i: (0, i)),
        ),
        out_shape=(
            jax.ShapeDtypeStruct((T, INPUT_DIM), jnp.float32),
            jax.ShapeDtypeStruct((NUM_Q, T), jnp.int32),
        ),
    )(x_td, Win_qcd, Wout_qdc, emb2, esq_qk)
    return out_td, codes_qt
